# baseline (speedup 1.0000x reference)
import functools

import jax
import jax.numpy as jnp
from jax import lax
from jax.experimental import pallas as pl
from jax.experimental.pallas import tpu as pltpu
from jax.experimental.pallas import tpu_sc as plsc

B, S, D, E, FF, TOP_K = 2, 2048, 1024, 8, 4096, 2
N = B * S

TM_S = 128
NSLOT = TOP_K * N + E * TM_S
NT = NSLOT // TM_S

_SQRT_HALF = 0.7071067811865476


def _erf(x):
    a1, a2, a3, a4, a5 = (0.254829592, -0.284496736, 1.421413741,
                          -1.453152027, 1.061405429)
    p = 0.3275911
    s = jnp.sign(x)
    ax = jnp.abs(x)
    t = 1.0 / (1.0 + p * ax)
    poly = t * (a1 + t * (a2 + t * (a3 + t * (a4 + t * a5))))
    return s * (1.0 - poly * jnp.exp(-ax * ax))


def _gelu(x):
    c = 0.7978845608028654
    return 0.5 * x * (1.0 + jnp.tanh(c * (x + 0.044715 * x * x * x)))


def _router_kernel(x_ref, wg_ref, e1_ref, e2_ref, w1_ref, w2_ref,
                   r1_ref, r2_ref, cnt_ref, aux_ref, run_ref, psum_ref,
                   cnt1_ref):
    m = pl.program_id(0)
    nm = pl.num_programs(0)
    tm = x_ref.shape[0]
    logits = jnp.dot(x_ref[...], wg_ref[...],
                     preferred_element_type=jnp.float32)
    mx = jnp.max(logits, axis=1, keepdims=True)
    ex = jnp.exp(logits - mx)
    probs = ex / jnp.sum(ex, axis=1, keepdims=True)
    eidx = jax.lax.broadcasted_iota(jnp.int32, probs.shape, 1)
    p1 = jnp.max(probs, axis=1, keepdims=True)
    i1 = jnp.min(jnp.where(probs == p1, eidx, E), axis=1, keepdims=True)
    probs2 = jnp.where(eidx == i1, -jnp.inf, probs)
    p2 = jnp.max(probs2, axis=1, keepdims=True)
    i2 = jnp.min(jnp.where(probs2 == p2, eidx, E), axis=1, keepdims=True)
    denom = p1 + p2 + 1e-9
    e1_ref[...] = i1
    e2_ref[...] = i2
    w1_ref[...] = p1 / denom
    w2_ref[...] = p2 / denom

    @pl.when(m == 0)
    def _init():
        run_ref[...] = jnp.zeros_like(run_ref)
        psum_ref[...] = jnp.zeros_like(psum_ref)
        cnt1_ref[...] = jnp.zeros_like(cnt1_ref)

    oh1 = (eidx == i1).astype(jnp.float32)
    oh2 = (eidx == i2).astype(jnp.float32)
    row = jax.lax.broadcasted_iota(jnp.int32, (tm, tm), 0)
    col = jax.lax.broadcasted_iota(jnp.int32, (tm, tm), 1)
    ltri = (row > col).astype(jnp.float32)
    c1 = jax.lax.dot(ltri, oh1, precision=jax.lax.Precision.HIGHEST)
    c2 = jax.lax.dot(ltri, oh2, precision=jax.lax.Precision.HIGHEST)
    s1 = jnp.sum(oh1, axis=0, keepdims=True)
    s2 = jnp.sum(oh2, axis=0, keepdims=True)
    run = run_ref[...]
    r1 = jnp.sum(oh1 * (run + c1), axis=1, keepdims=True)
    r2 = jnp.sum(oh2 * (run + s1 + c2), axis=1, keepdims=True)
    r1_ref[...] = r1
    r2_ref[...] = r2
    run_ref[...] = run + s1 + s2

    psum_ref[...] += jnp.sum(probs, axis=0, keepdims=True)
    cnt1_ref[...] += s1

    @pl.when(m == nm - 1)
    def _fin():
        cnt_ref[...] = run_ref[...]
        f = cnt1_ref[...] / N
        pmean = psum_ref[...] / N
        aux_ref[...] = (E * jnp.sum(f * pmean)).reshape(1, 1)


def _dispatch_kernel(e1_ref, e2_ref, r1_ref, r2_ref, cnt_ref,
                     pos1_ref, pos2_ref, te_ref):
    cnt = cnt_ref[...]
    padded = jnp.ceil(cnt / TM_S) * TM_S
    erow = jax.lax.broadcasted_iota(jnp.int32, (E, E), 0)
    ecol = jax.lax.broadcasted_iota(jnp.int32, (E, E), 1)
    sut = (erow < ecol).astype(jnp.float32)
    offset = jax.lax.dot(padded, sut,
                         precision=jax.lax.Precision.HIGHEST)
    eidx = jax.lax.broadcasted_iota(jnp.int32, (N, E), 1)
    off1 = jnp.sum(jnp.where(eidx == e1_ref[...], offset, 0.0), axis=1,
                   keepdims=True)
    off2 = jnp.sum(jnp.where(eidx == e2_ref[...], offset, 0.0), axis=1,
                   keepdims=True)
    pos1_ref[...] = (off1 + r1_ref[...]).astype(jnp.int32)
    pos2_ref[...] = (off2 + r2_ref[...]).astype(jnp.int32)
    cum_end = offset + padded
    tile_base = (jax.lax.broadcasted_iota(jnp.int32, (NT, E), 0)
                 * TM_S).astype(jnp.float32)
    te = jnp.sum((cum_end <= tile_base).astype(jnp.int32), axis=1,
                 keepdims=True)
    te_ref[...] = jnp.minimum(te, E - 1)


_SC_CHUNK = 32


def _sc_scatter_body(x_hbm, pos1_hbm, pos2_hbm, xs_hbm,
                     idx_v, rows_v, sem):
    nc = 2
    wid = lax.axis_index("s") * nc + lax.axis_index("c")
    per_w = N // 32
    nchunks = per_w // _SC_CHUNK

    def chunk(i, carry):
        base = wid * per_w + i * _SC_CHUNK
        pltpu.sync_copy(x_hbm.at[pl.ds(base, _SC_CHUNK)], rows_v)
        pltpu.sync_copy(pos1_hbm.at[pl.ds(base, _SC_CHUNK)], idx_v)
        pltpu.async_copy(rows_v, xs_hbm.at[idx_v], sem).wait()
        pltpu.sync_copy(pos2_hbm.at[pl.ds(base, _SC_CHUNK)], idx_v)
        pltpu.async_copy(rows_v, xs_hbm.at[idx_v], sem).wait()
        return carry

    lax.fori_loop(0, nchunks, chunk, 0)


def _sc_gather_body(y_hbm, pos1_hbm, pos2_hbm, ya_hbm, yb_hbm,
                    idx_v, rows_v, sem):
    nc = 2
    wid = lax.axis_index("s") * nc + lax.axis_index("c")
    per_w = N // 32
    nchunks = per_w // _SC_CHUNK

    def chunk(i, carry):
        base = wid * per_w + i * _SC_CHUNK
        pltpu.sync_copy(pos1_hbm.at[pl.ds(base, _SC_CHUNK)], idx_v)
        pltpu.async_copy(y_hbm.at[idx_v], rows_v, sem).wait()
        pltpu.sync_copy(rows_v, ya_hbm.at[pl.ds(base, _SC_CHUNK)])
        pltpu.sync_copy(pos2_hbm.at[pl.ds(base, _SC_CHUNK)], idx_v)
        pltpu.async_copy(y_hbm.at[idx_v], rows_v, sem).wait()
        pltpu.sync_copy(rows_v, yb_hbm.at[pl.ds(base, _SC_CHUNK)])
        return carry

    lax.fori_loop(0, nchunks, chunk, 0)


def _ffn1_kernel(te_ref, xs_ref, w1_ref, b1_ref, h_ref):
    h = jnp.dot(xs_ref[...], w1_ref[0],
                preferred_element_type=jnp.float32) + b1_ref[0]
    h_ref[...] = h.astype(jnp.bfloat16)


def _ffn2_kernel(te_ref, h_ref, w2_ref, b2_ref, y_ref):
    g = _gelu(h_ref[...].astype(jnp.float32))
    y_ref[...] = jnp.dot(g, w2_ref[0],
                         preferred_element_type=jnp.float32) + b2_ref[0]


def _combine_kernel(ya_ref, yb_ref, w1_ref, w2_ref, out_ref):
    out_ref[...] = ya_ref[...] * w1_ref[...] + yb_ref[...] * w2_ref[...]


@jax.jit
def kernel(x, Wg, W1, b1, W2, b2):
    x_flat = x.reshape(N, D)
    TM = 512
    nm = N // TM

    e1, e2, w1t, w2t, r1, r2, cnt, aux = pl.pallas_call(
        _router_kernel,
        grid=(nm,),
        in_specs=[
            pl.BlockSpec((TM, D), lambda m: (m, 0)),
            pl.BlockSpec((D, E), lambda m: (0, 0)),
        ],
        out_specs=[
            pl.BlockSpec((TM, 1), lambda m: (m, 0)),
            pl.BlockSpec((TM, 1), lambda m: (m, 0)),
            pl.BlockSpec((TM, 1), lambda m: (m, 0)),
            pl.BlockSpec((TM, 1), lambda m: (m, 0)),
            pl.BlockSpec((TM, 1), lambda m: (m, 0)),
            pl.BlockSpec((TM, 1), lambda m: (m, 0)),
            pl.BlockSpec((1, E), lambda m: (0, 0)),
            pl.BlockSpec((1, 1), lambda m: (0, 0)),
        ],
        out_shape=[
            jax.ShapeDtypeStruct((N, 1), jnp.int32),
            jax.ShapeDtypeStruct((N, 1), jnp.int32),
            jax.ShapeDtypeStruct((N, 1), jnp.float32),
            jax.ShapeDtypeStruct((N, 1), jnp.float32),
            jax.ShapeDtypeStruct((N, 1), jnp.float32),
            jax.ShapeDtypeStruct((N, 1), jnp.float32),
            jax.ShapeDtypeStruct((1, E), jnp.float32),
            jax.ShapeDtypeStruct((1, 1), jnp.float32),
        ],
        scratch_shapes=[
            pltpu.VMEM((1, E), jnp.float32),
            pltpu.VMEM((1, E), jnp.float32),
            pltpu.VMEM((1, E), jnp.float32),
        ],
    )(x_flat, Wg)

    pos1, pos2, te = pl.pallas_call(
        _dispatch_kernel,
        grid=(1,),
        in_specs=[
            pl.BlockSpec((N, 1), lambda i: (0, 0)),
            pl.BlockSpec((N, 1), lambda i: (0, 0)),
            pl.BlockSpec((N, 1), lambda i: (0, 0)),
            pl.BlockSpec((N, 1), lambda i: (0, 0)),
            pl.BlockSpec((1, E), lambda i: (0, 0)),
        ],
        out_specs=[
            pl.BlockSpec((N, 1), lambda i: (0, 0)),
            pl.BlockSpec((N, 1), lambda i: (0, 0)),
            pl.BlockSpec((NT, 1), lambda i: (0, 0)),
        ],
        out_shape=[
            jax.ShapeDtypeStruct((N, 1), jnp.int32),
            jax.ShapeDtypeStruct((N, 1), jnp.int32),
            jax.ShapeDtypeStruct((NT, 1), jnp.int32),
        ],
    )(e1, e2, r1, r2, cnt)

    pos1_f = pos1.reshape(N)
    pos2_f = pos2.reshape(N)

    mesh = plsc.VectorSubcoreMesh(core_axis_name="c", subcore_axis_name="s")
    xs = pl.kernel(
        _sc_scatter_body,
        mesh=mesh,
        out_type=jax.ShapeDtypeStruct((NSLOT, D), jnp.float32),
        scratch_types=[
            pltpu.VMEM((_SC_CHUNK,), jnp.int32),
            pltpu.VMEM((_SC_CHUNK, D), jnp.float32),
            pltpu.SemaphoreType.DMA,
        ],
    )(x_flat, pos1_f, pos2_f)

    te_flat = te.reshape(NT)
    grid1 = pltpu.PrefetchScalarGridSpec(
        num_scalar_prefetch=1,
        grid=(NT,),
        in_specs=[
            pl.BlockSpec((TM_S, D), lambda i, te_r: (i, 0)),
            pl.BlockSpec((1, D, FF), lambda i, te_r: (te_r[i], 0, 0)),
            pl.BlockSpec((1, 1, FF), lambda i, te_r: (te_r[i], 0, 0)),
        ],
        out_specs=pl.BlockSpec((TM_S, FF), lambda i, te_r: (i, 0)),
    )
    h = pl.pallas_call(
        _ffn1_kernel,
        grid_spec=grid1,
        out_shape=jax.ShapeDtypeStruct((NSLOT, FF), jnp.bfloat16),
    )(te_flat, xs, W1, b1.reshape(E, 1, FF))

    grid2 = pltpu.PrefetchScalarGridSpec(
        num_scalar_prefetch=1,
        grid=(NT,),
        in_specs=[
            pl.BlockSpec((TM_S, FF), lambda i, te_r: (i, 0)),
            pl.BlockSpec((1, FF, D), lambda i, te_r: (te_r[i], 0, 0)),
            pl.BlockSpec((1, 1, D), lambda i, te_r: (te_r[i], 0, 0)),
        ],
        out_specs=pl.BlockSpec((TM_S, D), lambda i, te_r: (i, 0)),
    )
    y = pl.pallas_call(
        _ffn2_kernel,
        grid_spec=grid2,
        out_shape=jax.ShapeDtypeStruct((NSLOT, D), jnp.float32),
    )(te_flat, h, W2, b2.reshape(E, 1, D))

    ya, yb = pl.kernel(
        _sc_gather_body,
        mesh=mesh,
        out_type=[
            jax.ShapeDtypeStruct((N, D), jnp.float32),
            jax.ShapeDtypeStruct((N, D), jnp.float32),
        ],
        scratch_types=[
            pltpu.VMEM((_SC_CHUNK,), jnp.int32),
            pltpu.VMEM((_SC_CHUNK, D), jnp.float32),
            pltpu.SemaphoreType.DMA,
        ],
    )(y, pos1_f, pos2_f)

    out = pl.pallas_call(
        _combine_kernel,
        grid=(nm,),
        in_specs=[
            pl.BlockSpec((TM, D), lambda m: (m, 0)),
            pl.BlockSpec((TM, D), lambda m: (m, 0)),
            pl.BlockSpec((TM, 1), lambda m: (m, 0)),
            pl.BlockSpec((TM, 1), lambda m: (m, 0)),
        ],
        out_specs=pl.BlockSpec((TM, D), lambda m: (m, 0)),
        out_shape=jax.ShapeDtypeStruct((N, D), jnp.float32),
    )(ya, yb, w1t, w2t)

    return out.reshape(B, S, D), aux[0, 0]


# speedup vs baseline: 1.0523x; 1.0523x over previous
import functools

import jax
import jax.numpy as jnp
from jax import lax
from jax.experimental import pallas as pl
from jax.experimental.pallas import tpu as pltpu
from jax.experimental.pallas import tpu_sc as plsc

B, S, D, E, FF, TOP_K = 2, 2048, 1024, 8, 4096, 2
N = B * S

TM_S = 256
NSLOT = TOP_K * N + E * TM_S
NT = NSLOT // TM_S

_SQRT_HALF = 0.7071067811865476


def _erf(x):
    a1, a2, a3, a4, a5 = (0.254829592, -0.284496736, 1.421413741,
                          -1.453152027, 1.061405429)
    p = 0.3275911
    s = jnp.sign(x)
    ax = jnp.abs(x)
    t = 1.0 / (1.0 + p * ax)
    poly = t * (a1 + t * (a2 + t * (a3 + t * (a4 + t * a5))))
    return s * (1.0 - poly * jnp.exp(-ax * ax))


def _gelu(x):
    c = 0.7978845608028654
    return 0.5 * x * (1.0 + jnp.tanh(c * (x + 0.044715 * x * x * x)))


def _router_kernel(x_ref, wg_ref, pos1_ref, pos2_ref, te_ref, w1_ref,
                   w2_ref, aux_ref, e1_ref, e2_ref, r1_ref, r2_ref,
                   run_ref, psum_ref, cnt1_ref):
    m = pl.program_id(0)
    nm = pl.num_programs(0)
    tm = x_ref.shape[0]
    logits = jnp.dot(x_ref[...], wg_ref[...],
                     preferred_element_type=jnp.float32)
    mx = jnp.max(logits, axis=1, keepdims=True)
    ex = jnp.exp(logits - mx)
    probs = ex / jnp.sum(ex, axis=1, keepdims=True)
    eidx = jax.lax.broadcasted_iota(jnp.int32, probs.shape, 1)
    p1 = jnp.max(probs, axis=1, keepdims=True)
    i1 = jnp.min(jnp.where(probs == p1, eidx, E), axis=1, keepdims=True)
    probs2 = jnp.where(eidx == i1, -jnp.inf, probs)
    p2 = jnp.max(probs2, axis=1, keepdims=True)
    i2 = jnp.min(jnp.where(probs2 == p2, eidx, E), axis=1, keepdims=True)
    denom = p1 + p2 + 1e-9
    row0 = m * tm
    e1_ref[pl.ds(row0, tm), :] = i1
    e2_ref[pl.ds(row0, tm), :] = i2
    w1_ref[...] = p1 / denom
    w2_ref[...] = p2 / denom

    @pl.when(m == 0)
    def _init():
        run_ref[...] = jnp.zeros_like(run_ref)
        psum_ref[...] = jnp.zeros_like(psum_ref)
        cnt1_ref[...] = jnp.zeros_like(cnt1_ref)

    oh1 = (eidx == i1).astype(jnp.float32)
    oh2 = (eidx == i2).astype(jnp.float32)
    row = jax.lax.broadcasted_iota(jnp.int32, (tm, tm), 0)
    col = jax.lax.broadcasted_iota(jnp.int32, (tm, tm), 1)
    ltri = (row > col).astype(jnp.float32)
    c1 = jax.lax.dot(ltri, oh1, precision=jax.lax.Precision.HIGHEST)
    c2 = jax.lax.dot(ltri, oh2, precision=jax.lax.Precision.HIGHEST)
    s1 = jnp.sum(oh1, axis=0, keepdims=True)
    s2 = jnp.sum(oh2, axis=0, keepdims=True)
    run = run_ref[...]
    r1 = jnp.sum(oh1 * (run + c1), axis=1, keepdims=True)
    r2 = jnp.sum(oh2 * (run + s1 + c2), axis=1, keepdims=True)
    r1_ref[pl.ds(row0, tm), :] = r1
    r2_ref[pl.ds(row0, tm), :] = r2
    run_ref[...] = run + s1 + s2

    psum_ref[...] += jnp.sum(probs, axis=0, keepdims=True)
    cnt1_ref[...] += s1

    @pl.when(m == nm - 1)
    def _fin():
        f = cnt1_ref[...] / N
        pmean = psum_ref[...] / N
        aux_ref[...] = (E * jnp.sum(f * pmean)).reshape(1, 1)
        cnt = run_ref[...]
        padded = jnp.ceil(cnt / TM_S) * TM_S
        erow = jax.lax.broadcasted_iota(jnp.int32, (E, E), 0)
        ecol = jax.lax.broadcasted_iota(jnp.int32, (E, E), 1)
        sut = (erow < ecol).astype(jnp.float32)
        offset = jax.lax.dot(padded, sut,
                             precision=jax.lax.Precision.HIGHEST)
        teidx = jax.lax.broadcasted_iota(jnp.int32, (N, E), 1)
        off1 = jnp.sum(jnp.where(teidx == e1_ref[...], offset, 0.0),
                       axis=1, keepdims=True)
        off2 = jnp.sum(jnp.where(teidx == e2_ref[...], offset, 0.0),
                       axis=1, keepdims=True)
        pos1_ref[...] = (off1 + r1_ref[...]).astype(jnp.int32)
        pos2_ref[...] = (off2 + r2_ref[...]).astype(jnp.int32)
        cum_end = offset + padded
        tile_base = (jax.lax.broadcasted_iota(jnp.int32, (NT, E), 0)
                     * TM_S).astype(jnp.float32)
        te = jnp.sum((cum_end <= tile_base).astype(jnp.int32), axis=1,
                     keepdims=True)
        te_ref[...] = jnp.minimum(te, E - 1)


_SC_CHUNK = 32


def _sc_scatter_body(x_hbm, pos1_hbm, pos2_hbm, xs_hbm,
                     idx_v, rows_v, sem):
    nc = 2
    wid = lax.axis_index("s") * nc + lax.axis_index("c")
    per_w = N // 32
    nchunks = per_w // _SC_CHUNK

    def chunk(i, carry):
        base = wid * per_w + i * _SC_CHUNK
        pltpu.sync_copy(x_hbm.at[pl.ds(base, _SC_CHUNK)], rows_v)
        pltpu.sync_copy(pos1_hbm.at[pl.ds(base, _SC_CHUNK)], idx_v)
        pltpu.async_copy(rows_v, xs_hbm.at[idx_v], sem).wait()
        pltpu.sync_copy(pos2_hbm.at[pl.ds(base, _SC_CHUNK)], idx_v)
        pltpu.async_copy(rows_v, xs_hbm.at[idx_v], sem).wait()
        return carry

    lax.fori_loop(0, nchunks, chunk, 0)


def _sc_gather_body(y_hbm, pos1_hbm, pos2_hbm, ya_hbm, yb_hbm,
                    idx_v, rows_v, sem):
    nc = 2
    wid = lax.axis_index("s") * nc + lax.axis_index("c")
    per_w = N // 32
    nchunks = per_w // _SC_CHUNK

    def chunk(i, carry):
        base = wid * per_w + i * _SC_CHUNK
        pltpu.sync_copy(pos1_hbm.at[pl.ds(base, _SC_CHUNK)], idx_v)
        pltpu.async_copy(y_hbm.at[idx_v], rows_v, sem).wait()
        pltpu.sync_copy(rows_v, ya_hbm.at[pl.ds(base, _SC_CHUNK)])
        pltpu.sync_copy(pos2_hbm.at[pl.ds(base, _SC_CHUNK)], idx_v)
        pltpu.async_copy(y_hbm.at[idx_v], rows_v, sem).wait()
        pltpu.sync_copy(rows_v, yb_hbm.at[pl.ds(base, _SC_CHUNK)])
        return carry

    lax.fori_loop(0, nchunks, chunk, 0)


def _ffn1_kernel(te_ref, xs_ref, w1_ref, b1_ref, h_ref):
    h = jnp.dot(xs_ref[...], w1_ref[0],
                preferred_element_type=jnp.float32) + b1_ref[0]
    h_ref[...] = h.astype(jnp.bfloat16)


def _ffn2_kernel(te_ref, h_ref, w2_ref, b2_ref, y_ref):
    g = _gelu(h_ref[...].astype(jnp.float32))
    y_ref[...] = jnp.dot(g, w2_ref[0],
                         preferred_element_type=jnp.float32) + b2_ref[0]


def _combine_kernel(ya_ref, yb_ref, w1_ref, w2_ref, out_ref):
    out_ref[...] = ya_ref[...] * w1_ref[...] + yb_ref[...] * w2_ref[...]


@jax.jit
def kernel(x, Wg, W1, b1, W2, b2):
    x_flat = x.reshape(N, D)
    TM = 512
    nm = N // TM

    pos1, pos2, te, w1t, w2t, aux = pl.pallas_call(
        _router_kernel,
        grid=(nm,),
        in_specs=[
            pl.BlockSpec((TM, D), lambda m: (m, 0)),
            pl.BlockSpec((D, E), lambda m: (0, 0)),
        ],
        out_specs=[
            pl.BlockSpec((N, 1), lambda m: (0, 0)),
            pl.BlockSpec((N, 1), lambda m: (0, 0)),
            pl.BlockSpec((NT, 1), lambda m: (0, 0)),
            pl.BlockSpec((TM, 1), lambda m: (m, 0)),
            pl.BlockSpec((TM, 1), lambda m: (m, 0)),
            pl.BlockSpec((1, 1), lambda m: (0, 0)),
        ],
        out_shape=[
            jax.ShapeDtypeStruct((N, 1), jnp.int32),
            jax.ShapeDtypeStruct((N, 1), jnp.int32),
            jax.ShapeDtypeStruct((NT, 1), jnp.int32),
            jax.ShapeDtypeStruct((N, 1), jnp.float32),
            jax.ShapeDtypeStruct((N, 1), jnp.float32),
            jax.ShapeDtypeStruct((1, 1), jnp.float32),
        ],
        scratch_shapes=[
            pltpu.VMEM((N, 1), jnp.int32),
            pltpu.VMEM((N, 1), jnp.int32),
            pltpu.VMEM((N, 1), jnp.float32),
            pltpu.VMEM((N, 1), jnp.float32),
            pltpu.VMEM((1, E), jnp.float32),
            pltpu.VMEM((1, E), jnp.float32),
            pltpu.VMEM((1, E), jnp.float32),
        ],
    )(x_flat, Wg)

    pos1_f = pos1.reshape(N)
    pos2_f = pos2.reshape(N)

    mesh = plsc.VectorSubcoreMesh(core_axis_name="c", subcore_axis_name="s")
    xs = pl.kernel(
        _sc_scatter_body,
        mesh=mesh,
        out_type=jax.ShapeDtypeStruct((NSLOT, D), jnp.float32),
        scratch_types=[
            pltpu.VMEM((_SC_CHUNK,), jnp.int32),
            pltpu.VMEM((_SC_CHUNK, D), jnp.float32),
            pltpu.SemaphoreType.DMA,
        ],
    )(x_flat, pos1_f, pos2_f)

    te_flat = te.reshape(NT)
    grid1 = pltpu.PrefetchScalarGridSpec(
        num_scalar_prefetch=1,
        grid=(NT,),
        in_specs=[
            pl.BlockSpec((TM_S, D), lambda i, te_r: (i, 0)),
            pl.BlockSpec((1, D, FF), lambda i, te_r: (te_r[i], 0, 0)),
            pl.BlockSpec((1, 1, FF), lambda i, te_r: (te_r[i], 0, 0)),
        ],
        out_specs=pl.BlockSpec((TM_S, FF), lambda i, te_r: (i, 0)),
    )
    h = pl.pallas_call(
        _ffn1_kernel,
        grid_spec=grid1,
        out_shape=jax.ShapeDtypeStruct((NSLOT, FF), jnp.bfloat16),
    )(te_flat, xs, W1, b1.reshape(E, 1, FF))

    grid2 = pltpu.PrefetchScalarGridSpec(
        num_scalar_prefetch=1,
        grid=(NT,),
        in_specs=[
            pl.BlockSpec((TM_S, FF), lambda i, te_r: (i, 0)),
            pl.BlockSpec((1, FF, D), lambda i, te_r: (te_r[i], 0, 0)),
            pl.BlockSpec((1, 1, D), lambda i, te_r: (te_r[i], 0, 0)),
        ],
        out_specs=pl.BlockSpec((TM_S, D), lambda i, te_r: (i, 0)),
    )
    y = pl.pallas_call(
        _ffn2_kernel,
        grid_spec=grid2,
        out_shape=jax.ShapeDtypeStruct((NSLOT, D), jnp.float32),
    )(te_flat, h, W2, b2.reshape(E, 1, D))

    ya, yb = pl.kernel(
        _sc_gather_body,
        mesh=mesh,
        out_type=[
            jax.ShapeDtypeStruct((N, D), jnp.float32),
            jax.ShapeDtypeStruct((N, D), jnp.float32),
        ],
        scratch_types=[
            pltpu.VMEM((_SC_CHUNK,), jnp.int32),
            pltpu.VMEM((_SC_CHUNK, D), jnp.float32),
            pltpu.SemaphoreType.DMA,
        ],
    )(y, pos1_f, pos2_f)

    out = pl.pallas_call(
        _combine_kernel,
        grid=(nm,),
        in_specs=[
            pl.BlockSpec((TM, D), lambda m: (m, 0)),
            pl.BlockSpec((TM, D), lambda m: (m, 0)),
            pl.BlockSpec((TM, 1), lambda m: (m, 0)),
            pl.BlockSpec((TM, 1), lambda m: (m, 0)),
        ],
        out_specs=pl.BlockSpec((TM, D), lambda m: (m, 0)),
        out_shape=jax.ShapeDtypeStruct((N, D), jnp.float32),
    )(ya, yb, w1t, w2t)

    return out.reshape(B, S, D), aux[0, 0]
